# Initial kernel scaffold; baseline (speedup 1.0000x reference)
import jax
import jax.numpy as jnp
from jax.experimental import pallas as pl


def kernel(x, edge_index_r0, edge_index_r1, edge_index_r2, W1_r0, b1_r0, W1_r1, b1_r1, W1_r2, b1_r2, W2_r0, b2_r0, W2_r1, b2_r1, W2_r2, b2_r2):
    raise NotImplementedError("write your pallas kernel here")


# speedup vs baseline: 2.3378x; 2.3378x over previous
import functools

import jax
import jax.numpy as jnp
from jax import lax
from jax.experimental import pallas as pl
from jax.experimental.pallas import tpu as pltpu
from jax.experimental.pallas import tpu_sc as plsc

N = 50000
D = 128
E = 200000
RELS = 3

NT = 16
NC = 2
NACC = 50176
RPT = NACC // NT
CW = 128
SECW = 14
SEC = 7
JS = SEC * SECW
STRIPE = JS * CW
E_PAD = NT * STRIPE
RING = 4
NB = 49
BN = NACC // NB
ZTAIL = RPT - (RPT // CW) * CW


def _prep_edges(ei):
    pad = E_PAD - E
    src = jnp.concatenate([ei[0], jnp.full((pad,), N, jnp.int32)]).reshape(NT, JS, CW)
    dst = jnp.concatenate([ei[1], jnp.full((pad,), N, jnp.int32)]).reshape(NT, JS, CW)
    return src, dst


def _sc_degrees(srcs, dsts):
    mesh = plsc.VectorSubcoreMesh(core_axis_name="c", subcore_axis_name="s")

    @functools.partial(
        pl.kernel,
        out_type=jax.ShapeDtypeStruct((2, RELS, NT, NACC), jnp.float32),
        mesh=mesh,
        scratch_types=[
            pltpu.VMEM((NACC,), jnp.float32),
            pltpu.VMEM((JS, CW), jnp.int32),
        ],
        compiler_params=pltpu.CompilerParams(
            use_tc_tiling_on_sc=False, needs_layout_passes=False),
    )
    def k(s0, s1, s2, d0, d1, d2, out, cnt_v, ids_v):
        cid = lax.axis_index("c")
        t = lax.axis_index("s")
        ones = jnp.ones((16,), jnp.float32)
        for side, arrs in ((0, (s0, s1, s2)), (1, (d0, d1, d2))):
            @pl.when(cid == side)
            def _(side=side, arrs=arrs):
                for r in range(RELS):
                    def zbody(i, c):
                        cnt_v[pl.ds(i * 16, 16)] = jnp.zeros((16,), jnp.float32)
                        return c
                    lax.fori_loop(0, NACC // 16, zbody, 0)
                    pltpu.sync_copy(arrs[r].at[t], ids_v)

                    def abody(j, c):
                        for kk in range(8):
                            ids = ids_v[j, pl.ds(kk * 16, 16)]
                            plsc.addupdate_scatter(cnt_v, [ids], ones)
                        return c
                    lax.fori_loop(0, JS, abody, 0)
                    pltpu.sync_copy(cnt_v, out.at[side, r, t])

    return k(srcs[0], srcs[1], srcs[2], dsts[0], dsts[1], dsts[2])


def _sc_agg(tbl, srcs, dsts):
    mesh = plsc.VectorSubcoreMesh(core_axis_name="c", subcore_axis_name="s")

    @functools.partial(
        pl.kernel,
        out_type=jax.ShapeDtypeStruct((RELS, 4, NACC, 32), jnp.float32),
        mesh=mesh,
        scratch_types=[
            pltpu.VMEM_SHARED((NACC, 32), jnp.float32),
            pltpu.VMEM((SECW, CW), jnp.int32),
            pltpu.VMEM((SECW, CW), jnp.int32),
            [pltpu.VMEM((CW, 32), jnp.float32) for _ in range(RING)],
            [pltpu.SemaphoreType.DMA for _ in range(RING)],
            [pltpu.SemaphoreType.DMA for _ in range(RING)],
        ],
        compiler_params=pltpu.CompilerParams(use_tc_tiling_on_sc=False),
    )
    def k(tbl_h, s0, s1, s2, d0, d1, d2, out, acc, srcv, dstv, bufs, gsems, ssems):
        cid = lax.axis_index("c")
        t = lax.axis_index("s")
        base = t * RPT
        srcs_h = (s0, s1, s2)
        dsts_h = (d0, d1, d2)
        for r in range(RELS):
            for cc in range(2):
                chunk = cid * 2 + cc

                def zb(i, c):
                    bufs[0][i, pl.ds(0, 16)] = jnp.zeros((16,), jnp.float32)
                    bufs[0][i, pl.ds(16, 16)] = jnp.zeros((16,), jnp.float32)
                    return c
                lax.fori_loop(0, CW, zb, 0)
                for kz in range(RPT // CW):
                    pltpu.sync_copy(bufs[0], acc.at[pl.ds(base + kz * CW, CW)])
                pltpu.sync_copy(bufs[0].at[pl.ds(0, ZTAIL)],
                                acc.at[pl.ds(base + (RPT // CW) * CW, ZTAIL)])
                plsc.subcore_barrier()

                src_h = srcs_h[r]
                dst_h = dsts_h[r]

                def sec_body(sec, carry, r=r, chunk=chunk, src_h=src_h, dst_h=dst_h):
                    pltpu.sync_copy(src_h.at[t, pl.ds(sec * SECW, SECW)], srcv)
                    pltpu.sync_copy(dst_h.at[t, pl.ds(sec * SECW, SECW)], dstv)
                    gd = [None] * SECW
                    sd = [None] * SECW
                    for j in range(RING):
                        gd[j] = pltpu.async_copy(
                            tbl_h.at[r, chunk].at[srcv.at[j]],
                            bufs[j], gsems[j])
                    for j in range(SECW):
                        gd[j].wait()
                        sd[j] = pltpu.async_copy(
                            bufs[j % RING], acc.at[dstv.at[j]],
                            ssems[j % RING], add=True)
                        jn = j + RING
                        if jn < SECW:
                            sd[j].wait()
                            gd[jn] = pltpu.async_copy(
                                tbl_h.at[r, chunk].at[srcv.at[jn]],
                                bufs[jn % RING], gsems[jn % RING])
                    for j in range(SECW - RING, SECW):
                        sd[j].wait()
                    return carry
                lax.fori_loop(0, SEC, sec_body, 0)
                plsc.subcore_barrier()
                pltpu.sync_copy(acc.at[pl.ds(base, RPT)],
                                out.at[r, chunk, pl.ds(base, RPT)])

    return k(tbl, srcs[0], srcs[1], srcs[2], dsts[0], dsts[1], dsts[2])


def _scales(c_ref, side, r):
    cnt = jnp.sum(c_ref[side, r], axis=0)
    return lax.rsqrt(jnp.maximum(cnt, 1.0))


def _tc_xn(xp, pcnt):
    def body(x_ref, c_ref, o_ref):
        for r in range(RELS):
            xs = x_ref[...] * _scales(c_ref, 0, r)[:, None]
            for c in range(4):
                o_ref[r, c] = xs[:, c * 32:(c + 1) * 32]

    return pl.pallas_call(
        body,
        grid=(NB,),
        in_specs=[
            pl.BlockSpec((BN, D), lambda i: (i, 0)),
            pl.BlockSpec((2, RELS, NT, BN), lambda i: (0, 0, 0, i)),
        ],
        out_specs=pl.BlockSpec((RELS, 4, BN, 32), lambda i: (0, 0, i, 0)),
        out_shape=jax.ShapeDtypeStruct((RELS, 4, NACC, 32), jnp.float32),
    )(xp, pcnt)


def _tc_layer1(agg, pcnt, W1s, b1s):
    def body(a_ref, c_ref, w_ref, b_ref, o_ref):
        h = jnp.zeros((BN, D), jnp.float32)
        for r in range(RELS):
            mm = jnp.zeros((BN, D), jnp.float32)
            w = w_ref[r]
            for c in range(4):
                mm = mm + jnp.dot(a_ref[r, c], w[c * 32:(c + 1) * 32, :],
                                  preferred_element_type=jnp.float32)
            h = h + mm * _scales(c_ref, 1, r)[:, None] + b_ref[r][None, :]
        h = jnp.maximum(h, 0.0)
        for r in range(RELS):
            hs = h * _scales(c_ref, 0, r)[:, None]
            for c in range(4):
                o_ref[r, c] = hs[:, c * 32:(c + 1) * 32]

    return pl.pallas_call(
        body,
        grid=(NB,),
        in_specs=[
            pl.BlockSpec((RELS, 4, BN, 32), lambda i: (0, 0, i, 0)),
            pl.BlockSpec((2, RELS, NT, BN), lambda i: (0, 0, 0, i)),
            pl.BlockSpec((RELS, D, D), lambda i: (0, 0, 0)),
            pl.BlockSpec((RELS, D), lambda i: (0, 0)),
        ],
        out_specs=pl.BlockSpec((RELS, 4, BN, 32), lambda i: (0, 0, i, 0)),
        out_shape=jax.ShapeDtypeStruct((RELS, 4, NACC, 32), jnp.float32),
    )(agg, pcnt, W1s, b1s)


def _tc_layer2(agg, pcnt, W2s, b2s):
    def body(a_ref, c_ref, w_ref, b_ref, o_ref):
        h = jnp.zeros((BN, D), jnp.float32)
        for r in range(RELS):
            mm = jnp.zeros((BN, D), jnp.float32)
            w = w_ref[r]
            for c in range(4):
                mm = mm + jnp.dot(a_ref[r, c], w[c * 32:(c + 1) * 32, :],
                                  preferred_element_type=jnp.float32)
            h = h + mm * _scales(c_ref, 1, r)[:, None] + b_ref[r][None, :]
        o_ref[...] = h

    return pl.pallas_call(
        body,
        grid=(NB,),
        in_specs=[
            pl.BlockSpec((RELS, 4, BN, 32), lambda i: (0, 0, i, 0)),
            pl.BlockSpec((2, RELS, NT, BN), lambda i: (0, 0, 0, i)),
            pl.BlockSpec((RELS, D, D), lambda i: (0, 0, 0)),
            pl.BlockSpec((RELS, D), lambda i: (0, 0)),
        ],
        out_specs=pl.BlockSpec((BN, D), lambda i: (i, 0)),
        out_shape=jax.ShapeDtypeStruct((NACC, D), jnp.float32),
    )(agg, pcnt, W2s, b2s)


def kernel(x, edge_index_r0, edge_index_r1, edge_index_r2,
           W1_r0, b1_r0, W1_r1, b1_r1, W1_r2, b1_r2,
           W2_r0, b2_r0, W2_r1, b2_r1, W2_r2, b2_r2):
    srcs, dsts = [], []
    for e in (edge_index_r0, edge_index_r1, edge_index_r2):
        s, d = _prep_edges(e)
        srcs.append(s)
        dsts.append(d)
    xp = jnp.pad(x, ((0, NACC - N), (0, 0)))
    W1s = jnp.stack([W1_r0, W1_r1, W1_r2])
    b1s = jnp.stack([b1_r0, b1_r1, b1_r2])
    W2s = jnp.stack([W2_r0, W2_r1, W2_r2])
    b2s = jnp.stack([b2_r0, b2_r1, b2_r2])

    pcnt = _sc_degrees(srcs, dsts)
    xn = _tc_xn(xp, pcnt)
    agg1 = _sc_agg(xn, srcs, dsts)
    hn = _tc_layer1(agg1, pcnt, W1s, b1s)
    agg2 = _sc_agg(hn, srcs, dsts)
    out = _tc_layer2(agg2, pcnt, W2s, b2s)
    return out[:N]


# speedup vs baseline: 4.2001x; 1.7966x over previous
import functools

import jax
import jax.numpy as jnp
from jax import lax
from jax.experimental import pallas as pl
from jax.experimental.pallas import tpu as pltpu
from jax.experimental.pallas import tpu_sc as plsc

N = 50000
D = 128
E = 200000
RELS = 3

NT = 16
NC = 2
NACC = 50176
RPT = NACC // NT
CW = 128
SECW = 14
SEC = 7
JS = SEC * SECW
STRIPE = JS * CW
E_PAD = NT * STRIPE
RING = 4
NBUF = 6
NB = 49
BN = NACC // NB
ZTAIL = RPT - (RPT // CW) * CW


def _prep_edges(ei):
    pad = E_PAD - E
    src = jnp.concatenate([ei[0], jnp.full((pad,), N, jnp.int32)]).reshape(NT, JS, CW)
    dst = jnp.concatenate([ei[1], jnp.full((pad,), N, jnp.int32)]).reshape(NT, JS, CW)
    src4 = jnp.stack([4 * src + c for c in range(4)])
    return src, src4, dst


def _sc_degrees(srcs, dsts):
    mesh = plsc.VectorSubcoreMesh(core_axis_name="c", subcore_axis_name="s")

    @functools.partial(
        pl.kernel,
        out_type=jax.ShapeDtypeStruct((2, RELS, NT, NACC), jnp.float32),
        mesh=mesh,
        scratch_types=[
            pltpu.VMEM((NACC,), jnp.float32),
            pltpu.VMEM((JS, CW), jnp.int32),
        ],
        compiler_params=pltpu.CompilerParams(
            use_tc_tiling_on_sc=False, needs_layout_passes=False),
    )
    def k(s0, s1, s2, d0, d1, d2, out, cnt_v, ids_v):
        cid = lax.axis_index("c")
        t = lax.axis_index("s")
        ones = jnp.ones((16,), jnp.float32)
        for side, arrs in ((0, (s0, s1, s2)), (1, (d0, d1, d2))):
            @pl.when(cid == side)
            def _(side=side, arrs=arrs):
                for r in range(RELS):
                    def zbody(i, c):
                        cnt_v[pl.ds(i * 16, 16)] = jnp.zeros((16,), jnp.float32)
                        return c
                    lax.fori_loop(0, NACC // 16, zbody, 0)
                    pltpu.sync_copy(arrs[r].at[t], ids_v)

                    def abody(j, c):
                        for kk in range(8):
                            ids = ids_v[j, pl.ds(kk * 16, 16)]
                            plsc.addupdate_scatter(cnt_v, [ids], ones)
                        return c
                    lax.fori_loop(0, JS, abody, 0)
                    pltpu.sync_copy(cnt_v, out.at[side, r, t])

    return k(srcs[0], srcs[1], srcs[2], dsts[0], dsts[1], dsts[2])


def _sc_agg(tbl, srcs4, dsts):
    mesh = plsc.VectorSubcoreMesh(core_axis_name="c", subcore_axis_name="s")

    @functools.partial(
        pl.kernel,
        out_type=jax.ShapeDtypeStruct((RELS, NACC, D), jnp.float32),
        mesh=mesh,
        scratch_types=[
            pltpu.VMEM_SHARED((NACC, 32), jnp.float32),
            pltpu.VMEM((SECW, CW), jnp.int32),
            pltpu.VMEM((SECW, CW), jnp.int32),
            [pltpu.VMEM((CW, 32), jnp.float32) for _ in range(NBUF)],
            [pltpu.SemaphoreType.DMA for _ in range(RING)],
            [pltpu.SemaphoreType.DMA for _ in range(NBUF)],
        ],
        compiler_params=pltpu.CompilerParams(use_tc_tiling_on_sc=False),
    )
    def k(tbl_h, s0, s1, s2, d0, d1, d2, out, acc, srcv, dstv, bufs, gsems, ssems):
        cid = lax.axis_index("c")
        t = lax.axis_index("s")
        base = t * RPT
        srcs_h = (s0, s1, s2)
        dsts_h = (d0, d1, d2)
        for r in range(RELS):
            for cc in range(2):
                chunk = cid * 2 + cc

                def zb(i, c):
                    bufs[0][i, pl.ds(0, 16)] = jnp.zeros((16,), jnp.float32)
                    bufs[0][i, pl.ds(16, 16)] = jnp.zeros((16,), jnp.float32)
                    return c
                lax.fori_loop(0, CW, zb, 0)
                for kz in range(RPT // CW):
                    pltpu.sync_copy(bufs[0], acc.at[pl.ds(base + kz * CW, CW)])
                pltpu.sync_copy(bufs[0].at[pl.ds(0, ZTAIL)],
                                acc.at[pl.ds(base + (RPT // CW) * CW, ZTAIL)])
                plsc.subcore_barrier()

                src_h = srcs_h[r]
                dst_h = dsts_h[r]

                def sec_body(sec, carry, r=r, chunk=chunk, src_h=src_h, dst_h=dst_h):
                    pltpu.sync_copy(src_h.at[chunk, t, pl.ds(sec * SECW, SECW)], srcv)
                    pltpu.sync_copy(dst_h.at[t, pl.ds(sec * SECW, SECW)], dstv)
                    gd = [None] * SECW
                    sd = [None] * SECW
                    for j in range(RING):
                        gd[j] = pltpu.async_copy(
                            tbl_h.at[r].at[srcv.at[j]],
                            bufs[j % NBUF], gsems[j % RING])
                    for j in range(SECW):
                        gd[j].wait()
                        sd[j] = pltpu.async_copy(
                            bufs[j % NBUF], acc.at[dstv.at[j]],
                            ssems[j % NBUF], add=True)
                        jn = j + RING
                        if jn < SECW:
                            js = jn - NBUF
                            if js >= 0:
                                sd[js].wait()
                            gd[jn] = pltpu.async_copy(
                                tbl_h.at[r].at[srcv.at[jn]],
                                bufs[jn % NBUF], gsems[jn % RING])
                    for j in range(SECW - NBUF, SECW):
                        sd[j].wait()
                    return carry
                lax.fori_loop(0, SEC, sec_body, 0)
                plsc.subcore_barrier()
                pltpu.sync_copy(
                    acc.at[pl.ds(base, RPT)],
                    out.at[r, pl.ds(base, RPT), pl.ds(chunk * 32, 32)])

    return k(tbl, srcs4[0], srcs4[1], srcs4[2], dsts[0], dsts[1], dsts[2])


def _scales(c_ref, side, r):
    cnt = jnp.sum(c_ref[side, r], axis=0)
    return lax.rsqrt(jnp.maximum(cnt, 1.0))


def _tc_xn(xp, pcnt):
    def body(x_ref, c_ref, o_ref):
        for r in range(RELS):
            o_ref[r] = x_ref[...] * _scales(c_ref, 0, r)[:, None]

    return pl.pallas_call(
        body,
        grid=(NB,),
        in_specs=[
            pl.BlockSpec((BN, D), lambda i: (i, 0)),
            pl.BlockSpec((2, RELS, NT, BN), lambda i: (0, 0, 0, i)),
        ],
        out_specs=pl.BlockSpec((RELS, BN, D), lambda i: (0, i, 0)),
        out_shape=jax.ShapeDtypeStruct((RELS, NACC, D), jnp.float32),
    )(xp, pcnt)


def _tc_layer1(agg, pcnt, W1s, b1s):
    def body(a_ref, c_ref, w_ref, b_ref, o_ref):
        h = jnp.zeros((BN, D), jnp.float32)
        for r in range(RELS):
            mm = jnp.dot(a_ref[r], w_ref[r], preferred_element_type=jnp.float32)
            h = h + mm * _scales(c_ref, 1, r)[:, None] + b_ref[r][None, :]
        h = jnp.maximum(h, 0.0)
        for r in range(RELS):
            o_ref[r] = h * _scales(c_ref, 0, r)[:, None]

    return pl.pallas_call(
        body,
        grid=(NB,),
        in_specs=[
            pl.BlockSpec((RELS, BN, D), lambda i: (0, i, 0)),
            pl.BlockSpec((2, RELS, NT, BN), lambda i: (0, 0, 0, i)),
            pl.BlockSpec((RELS, D, D), lambda i: (0, 0, 0)),
            pl.BlockSpec((RELS, D), lambda i: (0, 0)),
        ],
        out_specs=pl.BlockSpec((RELS, BN, D), lambda i: (0, i, 0)),
        out_shape=jax.ShapeDtypeStruct((RELS, NACC, D), jnp.float32),
    )(agg, pcnt, W1s, b1s)


def _tc_layer2(agg, pcnt, W2s, b2s):
    def body(a_ref, c_ref, w_ref, b_ref, o_ref):
        h = jnp.zeros((BN, D), jnp.float32)
        for r in range(RELS):
            mm = jnp.dot(a_ref[r], w_ref[r], preferred_element_type=jnp.float32)
            h = h + mm * _scales(c_ref, 1, r)[:, None] + b_ref[r][None, :]
        o_ref[...] = h

    return pl.pallas_call(
        body,
        grid=(NB,),
        in_specs=[
            pl.BlockSpec((RELS, BN, D), lambda i: (0, i, 0)),
            pl.BlockSpec((2, RELS, NT, BN), lambda i: (0, 0, 0, i)),
            pl.BlockSpec((RELS, D, D), lambda i: (0, 0, 0)),
            pl.BlockSpec((RELS, D), lambda i: (0, 0)),
        ],
        out_specs=pl.BlockSpec((BN, D), lambda i: (i, 0)),
        out_shape=jax.ShapeDtypeStruct((NACC, D), jnp.float32),
    )(agg, pcnt, W2s, b2s)


def kernel(x, edge_index_r0, edge_index_r1, edge_index_r2,
           W1_r0, b1_r0, W1_r1, b1_r1, W1_r2, b1_r2,
           W2_r0, b2_r0, W2_r1, b2_r1, W2_r2, b2_r2):
    srcs, srcs4, dsts = [], [], []
    for e in (edge_index_r0, edge_index_r1, edge_index_r2):
        s, s4, d = _prep_edges(e)
        srcs.append(s)
        srcs4.append(s4)
        dsts.append(d)
    xp = jnp.pad(x, ((0, NACC - N), (0, 0)))
    W1s = jnp.stack([W1_r0, W1_r1, W1_r2])
    b1s = jnp.stack([b1_r0, b1_r1, b1_r2])
    W2s = jnp.stack([W2_r0, W2_r1, W2_r2])
    b2s = jnp.stack([b2_r0, b2_r1, b2_r2])

    pcnt = _sc_degrees(srcs, dsts)
    xn = _tc_xn(xp, pcnt)
    agg1 = _sc_agg(xn.reshape(RELS, 4 * NACC, 32), srcs4, dsts)
    hn = _tc_layer1(agg1, pcnt, W1s, b1s)
    agg2 = _sc_agg(hn.reshape(RELS, 4 * NACC, 32), srcs4, dsts)
    out = _tc_layer2(agg2, pcnt, W2s, b2s)
    return out[:N]
